# baseline (speedup 1.0000x reference)
import functools

import jax
import jax.numpy as jnp
from jax import lax
from jax.experimental import pallas as pl
from jax.experimental.pallas import tpu as pltpu
from jax.experimental.pallas import tpu_sc as plsc

N_NODES = 10000
D = 128
DH = D // 2
E_EDGES = 320000
NC = 2
NS = 16
L = 16
CSZ = 128
CH = 162
CHH = CH // 2
SB = 9
NSB = CH // SB
EPW = CH * CSZ
EPAD = EPW * NS
NPAD = 10016
RPT = NPAD // NS
TPT = N_NODES // NS

_mesh = plsc.VectorSubcoreMesh(core_axis_name="c", subcore_axis_name="s")


@functools.partial(
    pl.kernel,
    mesh=_mesh,
    compiler_params=pltpu.CompilerParams(use_tc_tiling_on_sc=False),
    out_type=[
        jax.ShapeDtypeStruct((NC, NPAD, DH), jnp.float32),
        jax.ShapeDtypeStruct((NC, NPAD, L), jnp.float32),
    ],
    scratch_types=[
        pltpu.VMEM((2 * SB, CSZ), jnp.int32),
        pltpu.VMEM((2 * SB, CSZ), jnp.int32),
        [pltpu.VMEM((CSZ, DH), jnp.float32)] * 3,
        pltpu.VMEM((CSZ, L), jnp.float32),
        pltpu.VMEM_SHARED((N_NODES, DH), jnp.float32),
        pltpu.VMEM_SHARED((NPAD, DH), jnp.float32),
        pltpu.VMEM_SHARED((NPAD, L), jnp.float32),
        [pltpu.SemaphoreType.DMA] * 3,
        [pltpu.SemaphoreType.DMA] * 3,
        pltpu.SemaphoreType.DMA,
    ],
)
def _sc_aggregate(feat2_hbm, src_hbm, dst_hbm, z64_hbm, z16_hbm, ones_hbm,
                  acc_out, deg_out, src_v, dst_v, bufs, ones_v,
                  table_sh, acc_sh, deg_sh, gsem, ssem, isem):
    cid = lax.axis_index("c")
    sid = lax.axis_index("s")

    tb = sid * TPT
    pltpu.sync_copy(feat2_hbm.at[cid, pl.ds(tb, TPT)], table_sh.at[pl.ds(tb, TPT)])
    pltpu.sync_copy(src_hbm.at[sid, pl.ds(0, 2 * SB)], src_v)
    pltpu.sync_copy(dst_hbm.at[sid, pl.ds(0, 2 * SB)], dst_v)
    pltpu.sync_copy(ones_hbm, ones_v)
    base = sid * RPT
    pltpu.sync_copy(z64_hbm, acc_sh.at[pl.ds(base, RPT)])
    pltpu.sync_copy(z16_hbm, deg_sh.at[pl.ds(base, RPT)])
    plsc.subcore_barrier()

    def g_start(row, b):
        pltpu.async_copy(table_sh.at[src_v.at[row]], bufs[b], gsem[b])

    def g_wait(row, b):
        pltpu.make_async_copy(table_sh.at[src_v.at[row]], bufs[b], gsem[b]).wait()

    def s_start(row, b):
        pltpu.async_copy(bufs[b], acc_sh.at[dst_v.at[row]], ssem[b], add=True)

    def s_wait(row, b):
        pltpu.make_async_copy(bufs[b], acc_sh.at[dst_v.at[row]], ssem[b]).wait()

    def i_start(j2, h2):
        pltpu.async_copy(src_hbm.at[sid, pl.ds(j2, SB)],
                         src_v.at[pl.ds(h2 * SB, SB)], isem)
        pltpu.async_copy(dst_hbm.at[sid, pl.ds(j2, SB)],
                         dst_v.at[pl.ds(h2 * SB, SB)], isem)

    def i_wait(j2, h2):
        pltpu.make_async_copy(src_hbm.at[sid, pl.ds(j2, SB)],
                              src_v.at[pl.ds(h2 * SB, SB)], isem).wait()
        pltpu.make_async_copy(dst_hbm.at[sid, pl.ds(j2, SB)],
                              dst_v.at[pl.ds(h2 * SB, SB)], isem).wait()

    def chunk_work(c, q, row, drow, nrow, drain, prefetch):
        b = q % 3
        g_wait(row, b)
        if drain:
            s_wait(drow, (q - 2) % 3)
        if prefetch:
            g_start(nrow, (q + 1) % 3)
        s_start(row, b)

        do_deg = jnp.logical_or(jnp.logical_and(cid == 0, c < CHH),
                                jnp.logical_and(cid == 1, c >= CHH))

        @pl.when(do_deg)
        def _():
            pltpu.sync_copy(ones_v, deg_sh.at[dst_v.at[row]], add=True)

    def superblock(j, h, first, last):
        hb = h * SB
        h2b = (1 - h) * SB
        for q in range(SB):
            if q == SB - 1 and not (first or last):
                i_wait(j + SB, 1 - h)
            nrow = hb + q + 1 if q < SB - 1 else h2b
            drow = hb + q - 2 if q >= 2 else h2b + SB + q - 2
            chunk_work(j + q, q, hb + q, drow, nrow,
                       not (first and q < 2),
                       not (last and q == SB - 1))
            if q == 1 and not (first or last):
                i_start(j + SB, 1 - h)

    g_start(0, 0)
    superblock(0, 0, True, False)

    @pl.loop(SB, CH - SB, step=SB)
    def _(j):
        h = lax.rem(lax.div(j, SB), 2)
        superblock(j, h, False, False)

    superblock(CH - SB, (NSB - 1) % 2, False, True)
    s_wait(SB + SB - 2, (SB - 2) % 3)
    s_wait(SB + SB - 1, (SB - 1) % 3)

    plsc.subcore_barrier()
    pltpu.sync_copy(acc_sh.at[pl.ds(base, RPT)], acc_out.at[cid, pl.ds(base, RPT)])
    pltpu.sync_copy(deg_sh.at[pl.ds(base, RPT)], deg_out.at[cid, pl.ds(base, RPT)])


_RB = 2000


def _tc_self_body(feat_ref, ws_ref, b_ref, o_ref):
    o_ref[...] = (
        jnp.dot(feat_ref[...], ws_ref[...], preferred_element_type=jnp.float32)
        + b_ref[...]
    )


def _tc_self(feat, ws_t, b_row):
    return pl.pallas_call(
        _tc_self_body,
        grid=(N_NODES // _RB,),
        in_specs=[
            pl.BlockSpec((_RB, D), lambda r: (r, 0)),
            pl.BlockSpec((D, D), lambda r: (0, 0)),
            pl.BlockSpec((1, D), lambda r: (0, 0)),
        ],
        out_specs=pl.BlockSpec((_RB, D), lambda r: (r, 0)),
        out_shape=jax.ShapeDtypeStruct((N_NODES, D), jnp.float32),
    )(feat, ws_t, b_row)


def _tc_body(self_ref, acc_ref, degp_ref, wn_ref, o_ref):
    acc = jnp.concatenate([acc_ref[0], acc_ref[1]], axis=-1)
    deg = degp_ref[0, :, 0:1] + degp_ref[1, :, 0:1]
    deg = jnp.maximum(deg, 1.0)
    h_neigh = acc / deg
    o_ref[...] = (
        jnp.dot(h_neigh, wn_ref[...], preferred_element_type=jnp.float32)
        + self_ref[...]
    )


def _tc_combine(self_term, acc, degp, wn_t):
    return pl.pallas_call(
        _tc_body,
        grid=(N_NODES // _RB,),
        in_specs=[
            pl.BlockSpec((_RB, D), lambda r: (r, 0)),
            pl.BlockSpec((NC, _RB, DH), lambda r: (0, r, 0)),
            pl.BlockSpec((NC, _RB, L), lambda r: (0, r, 0)),
            pl.BlockSpec((D, D), lambda r: (0, 0)),
        ],
        out_specs=pl.BlockSpec((_RB, D), lambda r: (r, 0)),
        out_shape=jax.ShapeDtypeStruct((N_NODES, D), jnp.float32),
    )(self_term, acc, degp, wn_t)


@jax.jit
def kernel(feat, edge_index, W_neigh, W_self, b_self):
    src = edge_index[0]
    dst = edge_index[1]
    pad = EPAD - E_EDGES
    src_p = jnp.concatenate([src, jnp.zeros((pad,), jnp.int32)])
    dst_p = jnp.concatenate([dst, jnp.full((pad,), N_NODES, jnp.int32)])
    src_t = src_p.reshape(NS, CH, CSZ)
    dst_t = dst_p.reshape(NS, CH, CSZ)
    feat2 = jnp.stack([feat[:, :DH], feat[:, DH:]])
    z64 = jnp.zeros((RPT, DH), jnp.float32)
    z16 = jnp.zeros((RPT, L), jnp.float32)
    ones16 = jnp.ones((CSZ, L), jnp.float32)
    self_term = _tc_self(feat, W_self.T, b_self.reshape(1, D))
    acc, degp = _sc_aggregate(feat2, src_t, dst_t, z64, z16, ones16)
    return _tc_combine(self_term, acc, degp, W_neigh.T)


# speedup vs baseline: 1.0132x; 1.0132x over previous
import functools

import jax
import jax.numpy as jnp
from jax import lax
from jax.experimental import pallas as pl
from jax.experimental.pallas import tpu as pltpu
from jax.experimental.pallas import tpu_sc as plsc

N_NODES = 10000
D = 128
DH = D // 2
E_EDGES = 320000
NC = 2
NS = 16
L = 16
CSZ = 128
CH = 160
CHH = CH // 2
SB = 8
NSB = CH // SB
EPW = CH * CSZ
EPAD = EPW * NS
NPAD = 10016
RPT = NPAD // NS
TPT = N_NODES // NS

_mesh = plsc.VectorSubcoreMesh(core_axis_name="c", subcore_axis_name="s")


@functools.partial(
    pl.kernel,
    mesh=_mesh,
    compiler_params=pltpu.CompilerParams(use_tc_tiling_on_sc=False),
    out_type=[
        jax.ShapeDtypeStruct((NC, NPAD, DH), jnp.float32),
        jax.ShapeDtypeStruct((NC, NPAD, L), jnp.float32),
    ],
    scratch_types=[
        pltpu.VMEM((2 * SB, CSZ), jnp.int32),
        pltpu.VMEM((2 * SB, CSZ), jnp.int32),
        pltpu.VMEM((CSZ, DH), jnp.float32),
        pltpu.VMEM((CSZ, DH), jnp.float32),
        pltpu.VMEM((CSZ, L), jnp.float32),
        pltpu.VMEM_SHARED((N_NODES, DH), jnp.float32),
        pltpu.VMEM_SHARED((NPAD, DH), jnp.float32),
        pltpu.VMEM_SHARED((NPAD, L), jnp.float32),
        pltpu.SemaphoreType.DMA,
        pltpu.SemaphoreType.DMA,
        pltpu.SemaphoreType.DMA,
        pltpu.SemaphoreType.DMA,
    ],
)
def _sc_aggregate(feat2_hbm, src_hbm, dst_hbm, z64_hbm, z16_hbm, ones_hbm,
                  acc_out, deg_out, src_v, dst_v, buf0, buf1, ones_v,
                  table_sh, acc_sh, deg_sh, gsem0, gsem1, ssem, isem):
    cid = lax.axis_index("c")
    sid = lax.axis_index("s")

    tb = sid * TPT
    pltpu.sync_copy(feat2_hbm.at[cid, pl.ds(tb, TPT)], table_sh.at[pl.ds(tb, TPT)])
    pltpu.sync_copy(src_hbm.at[sid, pl.ds(0, 2 * SB)], src_v)
    pltpu.sync_copy(dst_hbm.at[sid, pl.ds(0, 2 * SB)], dst_v)
    pltpu.sync_copy(ones_hbm, ones_v)
    base = sid * RPT
    pltpu.sync_copy(z64_hbm, acc_sh.at[pl.ds(base, RPT)])
    pltpu.sync_copy(z16_hbm, deg_sh.at[pl.ds(base, RPT)])
    plsc.subcore_barrier()

    def g_start(row, buf, sem):
        pltpu.async_copy(table_sh.at[src_v.at[row]], buf, sem)

    def g_wait(row, buf, sem):
        pltpu.make_async_copy(table_sh.at[src_v.at[row]], buf, sem).wait()

    def i_start(j2, h2):
        pltpu.async_copy(src_hbm.at[sid, pl.ds(j2, SB)],
                         src_v.at[pl.ds(h2 * SB, SB)], isem)
        pltpu.async_copy(dst_hbm.at[sid, pl.ds(j2, SB)],
                         dst_v.at[pl.ds(h2 * SB, SB)], isem)

    def i_wait(j2, h2):
        pltpu.make_async_copy(src_hbm.at[sid, pl.ds(j2, SB)],
                              src_v.at[pl.ds(h2 * SB, SB)], isem).wait()
        pltpu.make_async_copy(dst_hbm.at[sid, pl.ds(j2, SB)],
                              dst_v.at[pl.ds(h2 * SB, SB)], isem).wait()

    def chunk_work(c, row, bufa, sema, bufb, semb, nrow, prefetch):
        g_wait(row, bufa, sema)
        if prefetch:
            g_start(nrow, bufb, semb)
        pltpu.async_copy(bufa, acc_sh.at[dst_v.at[row]], ssem, add=True)

        do_deg = jnp.logical_or(jnp.logical_and(cid == 0, c < CHH),
                                jnp.logical_and(cid == 1, c >= CHH))

        @pl.when(do_deg)
        def _():
            pltpu.sync_copy(ones_v, deg_sh.at[dst_v.at[row]], add=True)

        pltpu.make_async_copy(bufa, acc_sh.at[dst_v.at[row]], ssem).wait()

    def superblock(j, h, first, last):
        hb = h * SB
        h2b = (1 - h) * SB
        for q in range(SB):
            bufa, sema = (buf0, gsem0) if q % 2 == 0 else (buf1, gsem1)
            bufb, semb = (buf1, gsem1) if q % 2 == 0 else (buf0, gsem0)
            if q == 0 and not (first or last):
                i_start(j + SB, 1 - h)
            if q == SB - 1 and not (first or last):
                i_wait(j + SB, 1 - h)
            nrow = hb + q + 1 if q < SB - 1 else h2b
            chunk_work(j + q, hb + q, bufa, sema, bufb, semb, nrow,
                       not (last and q == SB - 1))

    g_start(0, buf0, gsem0)
    superblock(0, 0, True, False)

    @pl.loop(SB, CH - SB, step=SB)
    def _(j):
        h = lax.rem(lax.div(j, SB), 2)
        superblock(j, h, False, False)

    superblock(CH - SB, (NSB - 1) % 2, False, True)

    plsc.subcore_barrier()
    pltpu.sync_copy(acc_sh.at[pl.ds(base, RPT)], acc_out.at[cid, pl.ds(base, RPT)])
    pltpu.sync_copy(deg_sh.at[pl.ds(base, RPT)], deg_out.at[cid, pl.ds(base, RPT)])


_RB = 2000


def _tc_body(feat_ref, acc_ref, degp_ref, wn_ref, ws_ref, b_ref, o_ref):
    acc = jnp.concatenate([acc_ref[0], acc_ref[1]], axis=-1)
    deg = degp_ref[0, :, 0:1] + degp_ref[1, :, 0:1]
    deg = jnp.maximum(deg, 1.0)
    h_neigh = acc / deg
    o_ref[...] = (
        jnp.dot(h_neigh, wn_ref[...], preferred_element_type=jnp.float32)
        + jnp.dot(feat_ref[...], ws_ref[...], preferred_element_type=jnp.float32)
        + b_ref[...]
    )


def _tc_combine(feat, acc, degp, wn_t, ws_t, b_row):
    return pl.pallas_call(
        _tc_body,
        grid=(N_NODES // _RB,),
        in_specs=[
            pl.BlockSpec((_RB, D), lambda r: (r, 0)),
            pl.BlockSpec((NC, _RB, DH), lambda r: (0, r, 0)),
            pl.BlockSpec((NC, _RB, L), lambda r: (0, r, 0)),
            pl.BlockSpec((D, D), lambda r: (0, 0)),
            pl.BlockSpec((D, D), lambda r: (0, 0)),
            pl.BlockSpec((1, D), lambda r: (0, 0)),
        ],
        out_specs=pl.BlockSpec((_RB, D), lambda r: (r, 0)),
        out_shape=jax.ShapeDtypeStruct((N_NODES, D), jnp.float32),
    )(feat, acc, degp, wn_t, ws_t, b_row)


@jax.jit
def kernel(feat, edge_index, W_neigh, W_self, b_self):
    src = edge_index[0]
    dst = edge_index[1]
    pad = EPAD - E_EDGES
    src_p = jnp.concatenate([src, jnp.zeros((pad,), jnp.int32)])
    dst_p = jnp.concatenate([dst, jnp.full((pad,), N_NODES, jnp.int32)])
    src_t = src_p.reshape(NS, CH, CSZ)
    dst_t = dst_p.reshape(NS, CH, CSZ)
    feat2 = jnp.stack([feat[:, :DH], feat[:, DH:]])
    z64 = jnp.zeros((RPT, DH), jnp.float32)
    z16 = jnp.zeros((RPT, L), jnp.float32)
    ones16 = jnp.ones((CSZ, L), jnp.float32)
    acc, degp = _sc_aggregate(feat2, src_t, dst_t, z64, z16, ones16)
    return _tc_combine(feat, acc, degp, W_neigh.T, W_self.T,
                       b_self.reshape(1, D))
